# baseline (speedup 1.0000x reference)
import jax
import jax.numpy as jnp
from jax.experimental import pallas as pl
from jax.experimental.pallas import tpu as pltpu

_BM = 2048
_BN = 512


def _moe_dense_kernel(task_ref, x_ref, w_ref, b_ref, o_ref):
    del task_ref
    o_ref[...] = (
        jnp.dot(x_ref[...], w_ref[0], preferred_element_type=jnp.float32)
        + b_ref[0, 0]
    )


def kernel(inputs, kernel, bias, task_idx):
    m, k = inputs.shape
    n_tasks, _, n = kernel.shape
    t = jnp.clip(jnp.asarray(task_idx, jnp.int32), 0, n_tasks - 1).reshape((1,))
    bias3 = bias.reshape(n_tasks, 1, n)
    out = pl.pallas_call(
        _moe_dense_kernel,
        grid_spec=pltpu.PrefetchScalarGridSpec(
            num_scalar_prefetch=1,
            grid=(m // _BM, n // _BN),
            in_specs=[
                pl.BlockSpec((_BM, k), lambda i, j, s: (i, 0)),
                pl.BlockSpec((1, k, _BN), lambda i, j, s: (s[0], 0, j)),
                pl.BlockSpec((1, 1, _BN), lambda i, j, s: (s[0], 0, j)),
            ],
            out_specs=pl.BlockSpec((_BM, _BN), lambda i, j, s: (i, j)),
        ),
        out_shape=jax.ShapeDtypeStruct((m, n), jnp.float32),
    )(t, inputs, kernel, bias3)
    return out


# speedup vs baseline: 1.2487x; 1.2487x over previous
import jax
import jax.numpy as jnp
from jax.experimental import pallas as pl
from jax.experimental.pallas import tpu as pltpu

_BM = 512


def _moe_dense_kernel(task_ref, x_ref, w_ref, b_ref, o_ref):
    del task_ref
    o_ref[...] = (
        jnp.dot(x_ref[...], w_ref[0], preferred_element_type=jnp.float32)
        + b_ref[0, 0]
    )


def kernel(inputs, kernel, bias, task_idx):
    m, k = inputs.shape
    n_tasks, _, n = kernel.shape
    t = jnp.clip(jnp.asarray(task_idx, jnp.int32), 0, n_tasks - 1).reshape((1,))
    bias3 = bias.reshape(n_tasks, 1, n)
    out = pl.pallas_call(
        _moe_dense_kernel,
        grid_spec=pltpu.PrefetchScalarGridSpec(
            num_scalar_prefetch=1,
            grid=(m // _BM,),
            in_specs=[
                pl.BlockSpec((_BM, k), lambda i, s: (i, 0)),
                pl.BlockSpec((1, k, n), lambda i, s: (s[0], 0, 0)),
                pl.BlockSpec((1, 1, n), lambda i, s: (s[0], 0, 0)),
            ],
            out_specs=pl.BlockSpec((_BM, n), lambda i, s: (i, 0)),
        ),
        out_shape=jax.ShapeDtypeStruct((m, n), jnp.float32),
    )(t, inputs, kernel, bias3)
    return out


# speedup vs baseline: 1.2489x; 1.0002x over previous
import jax
import jax.numpy as jnp
from jax.experimental import pallas as pl
from jax.experimental.pallas import tpu as pltpu

_BM = 512


def _moe_dense_kernel(task_ref, x_ref, w_ref, b_ref, o_ref):
    del task_ref
    o_ref[...] = (
        jnp.dot(x_ref[...], w_ref[0], preferred_element_type=jnp.float32)
        + b_ref[0, 0]
    )


def kernel(inputs, kernel, bias, task_idx):
    m, k = inputs.shape
    n_tasks, _, n = kernel.shape
    t = jnp.clip(jnp.asarray(task_idx, jnp.int32), 0, n_tasks - 1).reshape((1,))
    bias3 = bias.reshape(n_tasks, 1, n)
    out = pl.pallas_call(
        _moe_dense_kernel,
        grid_spec=pltpu.PrefetchScalarGridSpec(
            num_scalar_prefetch=1,
            grid=(m // _BM,),
            in_specs=[
                pl.BlockSpec((_BM, k), lambda i, s: (i, 0)),
                pl.BlockSpec((1, k, n), lambda i, s: (s[0], 0, 0)),
                pl.BlockSpec((1, 1, n), lambda i, s: (s[0], 0, 0)),
            ],
            out_specs=pl.BlockSpec((_BM, n), lambda i, s: (i, 0)),
        ),
        out_shape=jax.ShapeDtypeStruct((m, n), jnp.float32),
        compiler_params=pltpu.CompilerParams(
            dimension_semantics=("parallel",),
        ),
    )(t, inputs, kernel, bias3)
    return out


# speedup vs baseline: 1.2677x; 1.0151x over previous
import jax
import jax.numpy as jnp
from jax.experimental import pallas as pl
from jax.experimental.pallas import tpu as pltpu

_BM = 1024


def _moe_dense_kernel(task_ref, x_ref, w_ref, b_ref, o_ref):
    del task_ref
    o_ref[...] = (
        jnp.dot(x_ref[...], w_ref[0], preferred_element_type=jnp.float32)
        + b_ref[0, 0]
    )


def kernel(inputs, kernel, bias, task_idx):
    m, k = inputs.shape
    n_tasks, _, n = kernel.shape
    t = jnp.clip(jnp.asarray(task_idx, jnp.int32), 0, n_tasks - 1).reshape((1,))
    bias3 = bias.reshape(n_tasks, 1, n)
    out = pl.pallas_call(
        _moe_dense_kernel,
        grid_spec=pltpu.PrefetchScalarGridSpec(
            num_scalar_prefetch=1,
            grid=(m // _BM,),
            in_specs=[
                pl.BlockSpec((_BM, k), lambda i, s: (i, 0)),
                pl.BlockSpec((1, k, n), lambda i, s: (s[0], 0, 0)),
                pl.BlockSpec((1, 1, n), lambda i, s: (s[0], 0, 0)),
            ],
            out_specs=pl.BlockSpec((_BM, n), lambda i, s: (i, 0)),
        ),
        out_shape=jax.ShapeDtypeStruct((m, n), jnp.float32),
        compiler_params=pltpu.CompilerParams(
            dimension_semantics=("parallel",),
            vmem_limit_bytes=63 * 1024 * 1024,
        ),
    )(t, inputs, kernel, bias3)
    return out
